# Initial kernel scaffold; baseline (speedup 1.0000x reference)
import jax
import jax.numpy as jnp
from jax.experimental import pallas as pl


def kernel(_, obs1, obs2, W, b):
    raise NotImplementedError("write your pallas kernel here")


# speedup vs baseline: 244.2171x; 244.2171x over previous
import functools

import jax
import jax.numpy as jnp
from jax import lax
from jax.experimental import pallas as pl

N = 2048
K = 8
OUT_PER = 8
BR = 256


def _tc_body2(x1r, y1r, x2c, y2c, x2r, y2r, wt, b2, out_ref):
    i = pl.program_id(0)
    base = i * BR

    col = lax.broadcasted_iota(jnp.int32, (BR, N), 1)
    row = base + lax.broadcasted_iota(jnp.int32, (BR, N), 0)

    relx = x2r[...] - x2c[...]
    rely = y2r[...] - y2c[...]
    dist = jnp.sqrt(relx * relx + rely * rely)
    dist = jnp.where(col == row, jnp.inf, dist)

    vxr = x2r[...] - x1r[...]
    vyr = y2r[...] - y1r[...]
    ptab = jnp.concatenate([x2r[...], y2r[...], vxr, vyr], axis=0).T

    rowhot = (col == row).astype(jnp.float32)
    self4 = jnp.dot(rowhot, ptab, preferred_element_type=jnp.float32)

    for k in range(K):
        m = jnp.min(dist, axis=1, keepdims=True)
        cand = jnp.where(dist == m, col, N)
        idx = jnp.min(cand, axis=1, keepdims=True)
        onehot = (col == idx).astype(jnp.float32)
        feats = jnp.dot(onehot, ptab, preferred_element_type=jnp.float32)
        rel = feats - self4
        emb = jnp.maximum(
            jnp.dot(rel, wt[...], preferred_element_type=jnp.float32)
            + b2[...], 0.0)
        out_ref[:, k * OUT_PER:(k + 1) * OUT_PER] = emb
        if k != K - 1:
            dist = jnp.where(col == idx, jnp.inf, dist)


@functools.partial(jax.jit, static_argnames=("interpret",))
def _run(obs1, obs2, W, b, interpret=False):
    x1r = obs1[:, 0].reshape(1, N)
    y1r = obs1[:, 1].reshape(1, N)
    x2r = obs2[:, 0].reshape(1, N)
    y2r = obs2[:, 1].reshape(1, N)
    x2c = obs2[:, 0].reshape(N, 1)
    y2c = obs2[:, 1].reshape(N, 1)
    wt = W.T
    b2 = b.reshape(1, OUT_PER)

    grid = (N // BR,)
    full_row = pl.BlockSpec((1, N), lambda i: (0, 0))
    col_blk = pl.BlockSpec((BR, 1), lambda i: (i, 0))
    return pl.pallas_call(
        _tc_body2,
        grid=grid,
        in_specs=[
            full_row, full_row,
            col_blk, col_blk,
            full_row, full_row,
            pl.BlockSpec((4, OUT_PER), lambda i: (0, 0)),
            pl.BlockSpec((1, OUT_PER), lambda i: (0, 0)),
        ],
        out_specs=pl.BlockSpec((BR, K * OUT_PER), lambda i: (i, 0)),
        out_shape=jax.ShapeDtypeStruct((N, K * OUT_PER), jnp.float32),
        interpret=interpret,
    )(x1r, y1r, x2c, y2c, x2r, y2r, wt, b2)


def kernel(_, obs1, obs2, W, b):
    return _run(obs1, obs2, W, b)
